# baseline (speedup 1.0000x reference)
import functools

import jax
import jax.numpy as jnp
from jax import lax
from jax.experimental import pallas as pl
from jax.experimental.pallas import tpu as pltpu
from jax.experimental.pallas import tpu_sc as plsc

B = 64
D = 1024
HID = 1024
OUT = 1024
E = 64
K = 2

_LANE = 16
_NSUB = 16
_ROWS = B // _NSUB


def _gate_body(x_ref, wg_ref, bg_ref, h_ref):
    h_ref[...] = (jnp.dot(x_ref[...], wg_ref[...],
                          preferred_element_type=jnp.float32) + bg_ref[...])


def _sc_router_body(h_hbm, probs_hbm, g_hbm, idx_hbm,
                    h_v, p_v, g_v, act_v, all_v, idx_v, shared):
    cid = lax.axis_index("c")
    sid = lax.axis_index("s")

    @pl.when(cid == 0)
    def _route():
        base = sid * _ROWS
        pltpu.sync_copy(h_hbm.at[pl.ds(base, _ROWS)], h_v)
        for k in range(E // _LANE):
            act_v[pl.ds(k * _LANE, _LANE)] = jnp.zeros((_LANE,), jnp.float32)
        for r in range(_ROWS):
            hk = [h_v[r, pl.ds(k * _LANE, _LANE)] for k in range(E // _LANE)]
            m1 = functools.reduce(jnp.maximum, [jnp.max(h) for h in hk])
            cnt = sum(jnp.sum(jnp.where(h == m1, 1.0, 0.0)) for h in hk)
            m2 = functools.reduce(
                jnp.maximum,
                [jnp.max(jnp.where(h == m1, -jnp.inf, h)) for h in hk])
            kth = jnp.where(cnt >= 2.0, m1, m2)
            ek = [jnp.exp(h - m1) for h in hk]
            s_all = sum(jnp.sum(e) for e in ek)
            gn = [jnp.where(h >= kth, e, 0.0) for h, e in zip(hk, ek)]
            s_top = sum(jnp.sum(g) for g in gn)
            for k in range(E // _LANE):
                sl = pl.ds(k * _LANE, _LANE)
                p_v[r, sl] = ek[k] / s_all
                g_v[r, sl] = gn[k] / s_top
                act_v[sl] = jnp.maximum(
                    act_v[sl], jnp.where(hk[k] >= kth, 1.0, 0.0))
        pltpu.sync_copy(p_v, probs_hbm.at[pl.ds(base, _ROWS)])
        pltpu.sync_copy(g_v, g_hbm.at[pl.ds(base, _ROWS)])
        pltpu.sync_copy(act_v, shared.at[sid])
        plsc.subcore_barrier()

        @pl.when(sid == 0)
        def _compact():
            pltpu.sync_copy(shared, all_v)
            count = jnp.int32(0)
            last = jnp.float32(-1.0)
            pos = []
            act = []
            for k in range(E // _LANE):
                a = all_v[0, pl.ds(k * _LANE, _LANE)]
                for j in range(1, _NSUB):
                    a = jnp.maximum(a, all_v[j, pl.ds(k * _LANE, _LANE)])
                ai = jnp.where(a > 0.0, 1, 0).astype(jnp.int32)
                c = plsc.cumsum(ai) + count
                pos.append(c - 1)
                act.append(a > 0.0)
                count = count + jnp.sum(ai)
                iota_f = lax.iota(jnp.int32, _LANE).astype(jnp.float32)
                iota_f = iota_f + jnp.float32(k * _LANE)
                last = jnp.maximum(
                    last, jnp.max(jnp.where(a > 0.0, iota_f, -1.0)))
            last_i = last.astype(jnp.int32)
            for k in range(E // _LANE):
                idx_v[pl.ds(k * _LANE, _LANE)] = (
                    jnp.zeros((_LANE,), jnp.int32) + last_i)
            for k in range(E // _LANE):
                vals = lax.iota(jnp.int32, _LANE) + jnp.int32(k * _LANE)
                plsc.store_scatter(idx_v, [pos[k]], vals, mask=act[k])
            pltpu.sync_copy(idx_v, idx_hbm)


def _expert_body(idx_ref, x_ref, g_ref, w1_ref, b1_ref, w2_ref, b2_ref,
                 wf_ref, bf_ref, out_ref, acc_ref, xb_ref):
    i = pl.program_id(0)
    e = idx_ref[i]
    prev = idx_ref[jnp.maximum(i - 1, 0)]
    is_new = (i == 0) | (e != prev)

    @pl.when(i == 0)
    def _init():
        acc_ref[...] = jnp.zeros_like(acc_ref)
        xb_ref[...] = x_ref[...].astype(jnp.bfloat16)

    @pl.when(is_new)
    def _compute():
        w1 = w1_ref[0].astype(jnp.bfloat16)
        h1 = jnp.dot(xb_ref[...], w1, preferred_element_type=jnp.float32)
        h1 = jnp.maximum(h1 + b1_ref[0, 0], 0.0)
        w2 = w2_ref[0].astype(jnp.bfloat16)
        eo = jnp.dot(h1.astype(jnp.bfloat16), w2,
                     preferred_element_type=jnp.float32) + b2_ref[0, 0]
        lane = lax.broadcasted_iota(jnp.int32, (B, E), 1)
        gate = jnp.sum(jnp.where(lane == e, g_ref[...], 0.0), axis=1,
                       keepdims=True)
        acc_ref[...] += gate * eo

    @pl.when(i == E - 1)
    def _final():
        out_ref[...] = jnp.dot(acc_ref[...], wf_ref[...],
                               preferred_element_type=jnp.float32) + bf_ref[...]


def kernel(x_list, Wg, bg, W1, b1, W2, b2, Wf, bf):
    x = x_list.reshape(B, D)

    H = pl.pallas_call(
        _gate_body,
        out_shape=jax.ShapeDtypeStruct((B, E), jnp.float32),
    )(x, Wg, bg.reshape(1, E))

    sc_router = pl.kernel(
        _sc_router_body,
        out_type=(
            jax.ShapeDtypeStruct((B, E), jnp.float32),
            jax.ShapeDtypeStruct((B, E), jnp.float32),
            jax.ShapeDtypeStruct((E,), jnp.int32),
        ),
        mesh=plsc.VectorSubcoreMesh(core_axis_name="c", subcore_axis_name="s"),
        compiler_params=pltpu.CompilerParams(needs_layout_passes=False),
        scratch_types=[
            pltpu.VMEM((_ROWS, E), jnp.float32),
            pltpu.VMEM((_ROWS, E), jnp.float32),
            pltpu.VMEM((_ROWS, E), jnp.float32),
            pltpu.VMEM((E,), jnp.float32),
            pltpu.VMEM((_NSUB, E), jnp.float32),
            pltpu.VMEM((E,), jnp.int32),
            pltpu.VMEM_SHARED((_NSUB, E), jnp.float32),
        ],
    )
    probs, G, idx = sc_router(H)

    grid_spec = pltpu.PrefetchScalarGridSpec(
        num_scalar_prefetch=1,
        grid=(E,),
        in_specs=[
            pl.BlockSpec((B, D), lambda i, idx_ref: (0, 0)),
            pl.BlockSpec((B, E), lambda i, idx_ref: (0, 0)),
            pl.BlockSpec((1, D, HID), lambda i, idx_ref: (idx_ref[i], 0, 0)),
            pl.BlockSpec((1, 1, HID), lambda i, idx_ref: (idx_ref[i], 0, 0)),
            pl.BlockSpec((1, HID, HID), lambda i, idx_ref: (idx_ref[i], 0, 0)),
            pl.BlockSpec((1, 1, HID), lambda i, idx_ref: (idx_ref[i], 0, 0)),
            pl.BlockSpec((HID, OUT), lambda i, idx_ref: (0, 0)),
            pl.BlockSpec((1, OUT), lambda i, idx_ref: (0, 0)),
        ],
        out_specs=pl.BlockSpec((B, OUT), lambda i, idx_ref: (0, 0)),
        scratch_shapes=[
            pltpu.VMEM((B, HID), jnp.float32),
            pltpu.VMEM((B, D), jnp.bfloat16),
        ],
    )
    out = pl.pallas_call(
        _expert_body,
        grid_spec=grid_spec,
        out_shape=jax.ShapeDtypeStruct((B, OUT), jnp.float32),
    )(idx, x, G, W1, b1.reshape(E, 1, HID), W2, b2.reshape(E, 1, HID),
      Wf, bf.reshape(1, OUT))

    return (out, probs.reshape(1, B, E))
